# baseline (speedup 1.0000x reference)
import jax
import jax.numpy as jnp
from jax.experimental import pallas as pl
from jax.experimental.pallas import tpu as pltpu

_PAD = 0
_VMEM = 64 * 1024 * 1024
_BF = jnp.bfloat16


def _recurrence(gif_ref, gib_ref, whh_ref, bhn_ref, out_ref):
    T, TB, H3 = gif_ref.shape
    H = H3 // 3
    whh_f = whh_ref[0].astype(_BF)
    whh_b = whh_ref[1].astype(_BF)
    bhn_f = bhn_ref[0]
    bhn_b = bhn_ref[1]

    def gate(gi_t, gh, bhn, h):
        rz = jax.nn.sigmoid(gi_t[:, :2 * H] + gh[:, :2 * H])
        r = rz[:, :H]
        z = rz[:, H:]
        n = jnp.tanh(gi_t[:, 2 * H:] + r * (gh[:, 2 * H:] + bhn))
        return n + z * (h - n)

    hf = jnp.zeros((TB, H), jnp.float32)
    hb = jnp.zeros((TB, H), jnp.float32)
    hf16 = hf.astype(_BF)
    hb16 = hb.astype(_BF)
    for s in range(T):
        tb = T - 1 - s
        ghf = jnp.dot(hf16, whh_f, preferred_element_type=jnp.float32)
        ghb = jnp.dot(hb16, whh_b, preferred_element_type=jnp.float32)
        hf = gate(gif_ref[s], ghf, bhn_f, hf)
        hb = gate(gib_ref[tb], ghb, bhn_b, hb)
        hf16 = hf.astype(_BF)
        hb16 = hb.astype(_BF)
        out_ref[s, :, :H] = hf16
        out_ref[tb, :, H:] = hb16


def _fused_body(x_ref, mask_ref,
                wih0_ref, whh0_ref, bgi0_ref, bhn0_ref,
                wih1_ref, whh1_ref, bgi1_ref, bhn1_ref,
                ws1_ref, ws2_ref, fcw_ref, fcb_ref, pw_ref, pb_ref,
                pred_ref, attn_ref,
                gif_ref, gib_ref, out0_ref, out1_ref, inpT_ref):
    T, TB, I = x_ref.shape
    H = whh0_ref.shape[1]
    D2 = 2 * H
    hops = ws2_ref.shape[1]

    x16 = x_ref[...].astype(_BF).reshape(T * TB, I)
    gif_ref[...] = (jnp.dot(x16, wih0_ref[0].astype(_BF),
                            preferred_element_type=jnp.float32)
                    + bgi0_ref[0]).reshape(T, TB, 3 * H)
    gib_ref[...] = (jnp.dot(x16, wih0_ref[1].astype(_BF),
                            preferred_element_type=jnp.float32)
                    + bgi0_ref[1]).reshape(T, TB, 3 * H)
    _recurrence(gif_ref, gib_ref, whh0_ref, bhn0_ref, out0_ref)

    x1 = out0_ref[...].reshape(T * TB, D2)
    gif_ref[...] = (jnp.dot(x1, wih1_ref[0].astype(_BF),
                            preferred_element_type=jnp.float32)
                    + bgi1_ref[0]).reshape(T, TB, 3 * H)
    gib_ref[...] = (jnp.dot(x1, wih1_ref[1].astype(_BF),
                            preferred_element_type=jnp.float32)
                    + bgi1_ref[1]).reshape(T, TB, 3 * H)
    _recurrence(gif_ref, gib_ref, whh1_ref, bhn1_ref, out1_ref)

    inpT_ref[...] = jnp.swapaxes(out1_ref[...], 0, 1)

    inp = inpT_ref[...]
    inp2 = inp.reshape(TB * T, D2)
    hbar = jnp.tanh(jnp.dot(inp2, ws1_ref[...].astype(_BF),
                            preferred_element_type=jnp.float32))
    scores = jnp.dot(hbar.astype(_BF), ws2_ref[...].astype(_BF),
                     preferred_element_type=jnp.float32)
    alphas = jnp.swapaxes(scores.reshape(TB, T, hops), 1, 2)
    pen = alphas - 10000.0 * mask_ref[...]
    m = jnp.max(pen, axis=-1, keepdims=True)
    e = jnp.exp(pen - m)
    a = e / jnp.sum(e, axis=-1, keepdims=True)
    attn_ref[...] = a
    agg = jnp.einsum("bht,btd->bhd", a.astype(_BF), inp,
                     preferred_element_type=jnp.float32)
    flat = agg.reshape(TB, hops * D2)
    fc = jnp.tanh(jnp.dot(flat, fcw_ref[...],
                          preferred_element_type=jnp.float32) + fcb_ref[...])
    pred = jnp.dot(fc, pw_ref[...],
                   preferred_element_type=jnp.float32) + pb_ref[...]
    pred_ref[...] = pred


def _fold_bias(bih, bhh):
    H3 = bih.shape[-1]
    H = H3 // 3
    bgi = bih.at[:, :, :2 * H].add(bhh[:, :, :2 * H])
    bhn = bhh[:, :, 2 * H:]
    return bgi, bhn


def kernel(tokens, emb, gru0_wih, gru0_whh, gru0_bih, gru0_bhh,
           gru1_wih, gru1_whh, gru1_bih, gru1_bhh,
           ws1, ws2, fcw, fcb, pw, pb):
    T, B = tokens.shape
    H = gru0_whh.shape[1]
    I = emb.shape[1]
    A = ws1.shape[1]
    hops = ws2.shape[1]
    nfc = fcw.shape[1]
    ncls = pw.shape[1]
    TB = B // 2
    D2 = 2 * H

    x = emb[tokens]
    mask = (tokens.T == _PAD).astype(jnp.float32)[:, None, :]
    bgi0, bhn0 = _fold_bias(gru0_bih, gru0_bhh)
    bgi1, bhn1 = _fold_bias(gru1_bih, gru1_bhh)

    whole = lambda *shape: pl.BlockSpec(shape, lambda c: (0,) * len(shape))
    pred, attn = pl.pallas_call(
        _fused_body,
        out_shape=(jax.ShapeDtypeStruct((B, ncls), jnp.float32),
                   jax.ShapeDtypeStruct((B, hops, T), jnp.float32)),
        grid=(2,),
        in_specs=[
            pl.BlockSpec((T, TB, I), lambda c: (0, c, 0)),
            pl.BlockSpec((TB, 1, T), lambda c: (c, 0, 0)),
            whole(2, I, 3 * H),
            whole(2, H, 3 * H),
            whole(2, 1, 3 * H),
            whole(2, 1, H),
            whole(2, D2, 3 * H),
            whole(2, H, 3 * H),
            whole(2, 1, 3 * H),
            whole(2, 1, H),
            whole(D2, A),
            whole(A, hops),
            whole(hops * D2, nfc),
            whole(1, nfc),
            whole(nfc, ncls),
            whole(1, ncls),
        ],
        out_specs=(pl.BlockSpec((TB, ncls), lambda c: (c, 0)),
                   pl.BlockSpec((TB, hops, T), lambda c: (c, 0, 0))),
        scratch_shapes=[pltpu.VMEM((T, TB, 3 * H), jnp.float32),
                        pltpu.VMEM((T, TB, 3 * H), jnp.float32),
                        pltpu.VMEM((T, TB, D2), _BF),
                        pltpu.VMEM((T, TB, D2), _BF),
                        pltpu.VMEM((TB, T, D2), _BF)],
        compiler_params=pltpu.CompilerParams(
            dimension_semantics=("parallel",),
            vmem_limit_bytes=_VMEM),
    )(x, mask, gru0_wih, gru0_whh, bgi0, bhn0,
      gru1_wih, gru1_whh, bgi1, bhn1,
      ws1, ws2, fcw, fcb, pw, pb)
    return pred, attn


# speedup vs baseline: 1.2577x; 1.2577x over previous
import jax
import jax.numpy as jnp
from jax.experimental import pallas as pl
from jax.experimental.pallas import tpu as pltpu

_PAD = 0
_VMEM = 64 * 1024 * 1024
_BF = jnp.bfloat16


def _recurrence(gif_ref, gib_ref, whh_ref, bhn_ref, out_ref):
    T, TB, H3 = gif_ref.shape
    H = H3 // 3
    whh_f = whh_ref[0]
    whh_b = whh_ref[1]
    bhn_f = bhn_ref[0]
    bhn_b = bhn_ref[1]

    def gate(gi_t, gh, bhn, h):
        rz = jax.nn.sigmoid(gi_t[:, :2 * H] + gh[:, :2 * H])
        r = rz[:, :H]
        z = rz[:, H:]
        n = jnp.tanh(gi_t[:, 2 * H:] + r * (gh[:, 2 * H:] + bhn))
        return n + z * (h - n)

    hf = jnp.zeros((TB, H), jnp.float32)
    hb = jnp.zeros((TB, H), jnp.float32)
    hf16 = hf.astype(_BF)
    hb16 = hb.astype(_BF)
    for s in range(T):
        tb = T - 1 - s
        ghf = jnp.dot(hf16, whh_f, preferred_element_type=jnp.float32)
        ghb = jnp.dot(hb16, whh_b, preferred_element_type=jnp.float32)
        hf = gate(gif_ref[s], ghf, bhn_f, hf)
        hb = gate(gib_ref[tb], ghb, bhn_b, hb)
        hf16 = hf.astype(_BF)
        hb16 = hb.astype(_BF)
        out_ref[s, :, :H] = hf16
        out_ref[tb, :, H:] = hb16


def _fused_body(x_ref, mask_ref,
                wih0_ref, whh0_ref, bgi0_ref, bhn0_ref,
                wih1_ref, whh1_ref, bgi1_ref, bhn1_ref,
                ws1_ref, ws2_ref, fcw_ref, fcb_ref, pw_ref, pb_ref,
                pred_ref, attn_ref,
                gif_ref, gib_ref, out0_ref, out1_ref, inpT_ref):
    T, TB, I = x_ref.shape
    H = whh0_ref.shape[1]
    D2 = 2 * H
    hops = ws2_ref.shape[1]

    x16 = x_ref[...].reshape(T * TB, I)
    gif_ref[...] = (jnp.dot(x16, wih0_ref[0],
                            preferred_element_type=jnp.float32)
                    + bgi0_ref[0]).reshape(T, TB, 3 * H)
    gib_ref[...] = (jnp.dot(x16, wih0_ref[1],
                            preferred_element_type=jnp.float32)
                    + bgi0_ref[1]).reshape(T, TB, 3 * H)
    _recurrence(gif_ref, gib_ref, whh0_ref, bhn0_ref, out0_ref)

    x1 = out0_ref[...].reshape(T * TB, D2)
    gif_ref[...] = (jnp.dot(x1, wih1_ref[0],
                            preferred_element_type=jnp.float32)
                    + bgi1_ref[0]).reshape(T, TB, 3 * H)
    gib_ref[...] = (jnp.dot(x1, wih1_ref[1],
                            preferred_element_type=jnp.float32)
                    + bgi1_ref[1]).reshape(T, TB, 3 * H)
    _recurrence(gif_ref, gib_ref, whh1_ref, bhn1_ref, out1_ref)

    inpT_ref[...] = jnp.swapaxes(out1_ref[...], 0, 1)

    inp = inpT_ref[...]
    inp2 = inp.reshape(TB * T, D2)
    hbar = jnp.tanh(jnp.dot(inp2, ws1_ref[...].astype(_BF),
                            preferred_element_type=jnp.float32))
    scores = jnp.dot(hbar.astype(_BF), ws2_ref[...].astype(_BF),
                     preferred_element_type=jnp.float32)
    alphas = jnp.swapaxes(scores.reshape(TB, T, hops), 1, 2)
    pen = alphas - 10000.0 * mask_ref[...]
    m = jnp.max(pen, axis=-1, keepdims=True)
    e = jnp.exp(pen - m)
    a = e / jnp.sum(e, axis=-1, keepdims=True)
    attn_ref[...] = a
    agg = jnp.einsum("bht,btd->bhd", a.astype(_BF), inp,
                     preferred_element_type=jnp.float32)
    flat = agg.reshape(TB, hops * D2)
    fc = jnp.tanh(jnp.dot(flat, fcw_ref[...],
                          preferred_element_type=jnp.float32) + fcb_ref[...])
    pred = jnp.dot(fc, pw_ref[...],
                   preferred_element_type=jnp.float32) + pb_ref[...]
    pred_ref[...] = pred


def _fold_bias(bih, bhh):
    H3 = bih.shape[-1]
    H = H3 // 3
    bgi = bih.at[:, :, :2 * H].add(bhh[:, :, :2 * H])
    bhn = bhh[:, :, 2 * H:]
    return bgi, bhn


def kernel(tokens, emb, gru0_wih, gru0_whh, gru0_bih, gru0_bhh,
           gru1_wih, gru1_whh, gru1_bih, gru1_bhh,
           ws1, ws2, fcw, fcb, pw, pb):
    T, B = tokens.shape
    H = gru0_whh.shape[1]
    I = emb.shape[1]
    A = ws1.shape[1]
    hops = ws2.shape[1]
    nfc = fcw.shape[1]
    ncls = pw.shape[1]
    TB = B // 2
    D2 = 2 * H

    x = emb[tokens].astype(_BF)
    mask = (tokens.T == _PAD).astype(jnp.float32)[:, None, :]
    bgi0, bhn0 = _fold_bias(gru0_bih, gru0_bhh)
    bgi1, bhn1 = _fold_bias(gru1_bih, gru1_bhh)

    whole = lambda *shape: pl.BlockSpec(shape, lambda c: (0,) * len(shape))
    pred, attn = pl.pallas_call(
        _fused_body,
        out_shape=(jax.ShapeDtypeStruct((B, ncls), jnp.float32),
                   jax.ShapeDtypeStruct((B, hops, T), jnp.float32)),
        grid=(2,),
        in_specs=[
            pl.BlockSpec((T, TB, I), lambda c: (0, c, 0)),
            pl.BlockSpec((TB, 1, T), lambda c: (c, 0, 0)),
            whole(2, I, 3 * H),
            whole(2, H, 3 * H),
            whole(2, 1, 3 * H),
            whole(2, 1, H),
            whole(2, D2, 3 * H),
            whole(2, H, 3 * H),
            whole(2, 1, 3 * H),
            whole(2, 1, H),
            whole(D2, A),
            whole(A, hops),
            whole(hops * D2, nfc),
            whole(1, nfc),
            whole(nfc, ncls),
            whole(1, ncls),
        ],
        out_specs=(pl.BlockSpec((TB, ncls), lambda c: (c, 0)),
                   pl.BlockSpec((TB, hops, T), lambda c: (c, 0, 0))),
        scratch_shapes=[pltpu.VMEM((T, TB, 3 * H), jnp.float32),
                        pltpu.VMEM((T, TB, 3 * H), jnp.float32),
                        pltpu.VMEM((T, TB, D2), _BF),
                        pltpu.VMEM((T, TB, D2), _BF),
                        pltpu.VMEM((TB, T, D2), _BF)],
        compiler_params=pltpu.CompilerParams(
            dimension_semantics=("parallel",),
            vmem_limit_bytes=_VMEM),
    )(x, mask, gru0_wih.astype(_BF), gru0_whh.astype(_BF), bgi0, bhn0,
      gru1_wih.astype(_BF), gru1_whh.astype(_BF), bgi1, bhn1,
      ws1, ws2, fcw, fcb, pw, pb)
    return pred, attn
